# Initial kernel scaffold; baseline (speedup 1.0000x reference)
import jax
import jax.numpy as jnp
from jax.experimental import pallas as pl


def kernel(queries, keys, values, Wq, bq, Wk, bk, Wv, bv):
    raise NotImplementedError("write your pallas kernel here")


# speedup vs baseline: 453.6945x; 453.6945x over previous
import functools

import jax
import jax.numpy as jnp
import numpy as np
from jax.experimental import pallas as pl
from jax.experimental.pallas import tpu as pltpu

_L = 2048
_N1 = 2 * _L - 1
_N2 = 2 * _L - 2
_B = 2
_H = 12
_D = 64
_BH = _B * _H
_TOPK = 40
_KB = 256
_HIGHEST = jax.lax.Precision.HIGHEST


def _split3_np(x):
    import ml_dtypes
    bf = ml_dtypes.bfloat16
    x0 = x.astype(bf)
    r = x.astype(np.float32) - x0.astype(np.float32)
    x1 = r.astype(bf)
    x2 = (r - x1.astype(np.float32)).astype(bf)
    return x0, x1, x2


def _dft_constants():
    t = np.arange(_L, dtype=np.int64)
    outer = np.outer(t, t)
    ang1 = (outer % _N1).astype(np.float64) * (2.0 * np.pi / _N1)
    ang2 = (outer % _N2).astype(np.float64) * (2.0 * np.pi / _N2)
    cf = np.cos(ang1).astype(np.float32)
    sf = np.sin(ang1).astype(np.float32)
    ci = np.cos(ang2).astype(np.float32)
    si = np.sin(ang2).astype(np.float32)
    nb = _L // _KB
    cfs = np.stack([np.hstack([cf[:, i * _KB:(i + 1) * _KB],
                               sf[:, i * _KB:(i + 1) * _KB]])
                    for i in range(nb)])
    cis = np.stack([np.vstack([ci[i * _KB:(i + 1) * _KB],
                               si[i * _KB:(i + 1) * _KB]])
                    for i in range(nb)])
    return _split3_np(cfs), _split3_np(cis)


_CFS3, _CIS3 = _dft_constants()


def _split3(x):
    x0 = x.astype(jnp.bfloat16)
    r = x - x0.astype(jnp.float32)
    x1 = r.astype(jnp.bfloat16)
    x2 = (r - x1.astype(jnp.float32)).astype(jnp.bfloat16)
    return x0, x1, x2


def _dot_x6(a3, b3):
    d = functools.partial(jnp.dot, preferred_element_type=jnp.float32)
    a0, a1, a2 = a3
    b0, b1, b2 = b3
    s = d(a0, b2) + d(a1, b1) + d(a2, b0)
    s = s + (d(a0, b1) + d(a1, b0))
    return s + d(a0, b0)


def _proj_body(q_ref, k_ref, v_ref, w_ref, b_ref, qk_ref, vs_ref):
    w16 = w_ref[...].astype(jnp.bfloat16)
    xq = q_ref[0, 0, :, :].astype(jnp.bfloat16)
    xk = k_ref[0, 0, :, :].astype(jnp.bfloat16)
    xv = v_ref[0, 0, :, :].astype(jnp.bfloat16)
    d = functools.partial(jnp.dot, preferred_element_type=jnp.float32)
    qs = d(xq, w16[:, 0]) + b_ref[0]
    ks = d(xk, w16[:, 1]) + b_ref[1]
    vs = d(xv, w16[:, 2]) + b_ref[2]
    qk_ref[0, 0, 0, :] = qs
    qk_ref[1, 0, 0, :] = ks
    vs_ref[0, 0, :] = vs


def _score_body(qk_ref, cfs0_ref, cfs1_ref, cfs2_ref, cis0_ref, cis1_ref,
                cis2_ref, wn_ref, idx_ref, acc_ref):
    i = pl.program_id(0)

    @pl.when(i == 0)
    def _():
        acc_ref[...] = jnp.zeros_like(acc_ref)

    qk3 = _split3(qk_ref[...])
    reim = _dot_x6(qk3, (cfs0_ref[0], cfs1_ref[0], cfs2_ref[0]))
    re = reim[:, :_KB]
    im = -reim[:, _KB:]
    qr, kr = re[:_BH], re[_BH:]
    qi, ki = im[:_BH], im[_BH:]
    sr = qr * kr + qi * ki
    si = qi * kr - qr * ki
    gcol = i * _KB + jax.lax.broadcasted_iota(jnp.int32, (_BH, _KB), 1)
    cr = jnp.where((gcol == 0) | (gcol == _L - 1), 1.0, 2.0)
    cim = jnp.where(gcol == _L - 1, 0.0, cr)
    a2 = jnp.concatenate([sr * cr, -(si * cim)], axis=1)
    acc_ref[...] += _dot_x6(_split3(a2),
                            (cis0_ref[0], cis1_ref[0], cis2_ref[0]))

    @pl.when(i == pl.num_programs(0) - 1)
    def _():
        score = acc_ref[...] * (1.0 / (float(_N2) * float(_L)))
        iota = jax.lax.broadcasted_iota(jnp.int32, (_BH, _L), 1)
        vals = []
        idxs = []
        for _d in range(_TOPK):
            m = jnp.max(score, axis=1, keepdims=True)
            hit = score == m
            idx = jnp.min(jnp.where(hit, iota, _L), axis=1, keepdims=True)
            vals.append(m)
            idxs.append(idx)
            score = jnp.where(iota == idx, -jnp.inf, score)
        w = jnp.concatenate(vals, axis=1)
        e = jnp.exp(w - w[:, 0:1])
        wn_ref[...] = e / jnp.sum(e, axis=1, keepdims=True)
        idx_ref[...] = jnp.concatenate(idxs, axis=1)


def _combine_body(i_smem, w_smem, vs_ref, out_ref):
    row = pl.program_id(0)
    vrow = vs_ref[pl.ds(row, 1), :]
    acc = jnp.zeros((1, _L), jnp.float32)
    for d in range(_TOPK):
        st = i_smem[row * _TOPK + d]
        w = w_smem[row * _TOPK + d]
        acc = acc + pltpu.roll(vrow, -st, axis=1) * w
    out_ref[0] = acc


@jax.jit
def kernel(queries, keys, values, Wq, bq, Wk, bk, Wv, bv):
    qt = jnp.transpose(queries, (0, 2, 1, 3))
    kt = jnp.transpose(keys, (0, 2, 1, 3))
    vt = jnp.transpose(values, (0, 2, 1, 3))
    w3 = jnp.stack([Wq[0], Wk[0], Wv[0]], axis=1)
    b3 = jnp.concatenate([bq, bk, bv])

    in_map = lambda i: (i // _H, i % _H, 0, 0)
    qk2, vs = pl.pallas_call(
        _proj_body,
        grid=(_BH,),
        in_specs=[
            pl.BlockSpec((1, 1, _L, _D), in_map),
            pl.BlockSpec((1, 1, _L, _D), in_map),
            pl.BlockSpec((1, 1, _L, _D), in_map),
            pl.BlockSpec((_D, 3), lambda i: (0, 0)),
            pl.BlockSpec(memory_space=pltpu.SMEM),
        ],
        out_specs=[
            pl.BlockSpec((2, 1, 1, _L), lambda i: (0, i, 0, 0)),
            pl.BlockSpec((1, 1, _L), lambda i: (i, 0, 0)),
        ],
        out_shape=[
            jax.ShapeDtypeStruct((2, _BH, 1, _L), jnp.float32),
            jax.ShapeDtypeStruct((_BH, 1, _L), jnp.float32),
        ],
    )(qt, kt, vt, w3, b3)
    qk = qk2.reshape(2 * _BH, _L)
    vs = vs.reshape(_BH, _L)

    nsteps = _L // _KB
    fwd_spec = pl.BlockSpec((1, _L, 2 * _KB), lambda i: (i, 0, 0))
    inv_spec = pl.BlockSpec((1, 2 * _KB, _L), lambda i: (i, 0, 0))
    wn, idx = pl.pallas_call(
        _score_body,
        grid=(nsteps,),
        in_specs=[
            pl.BlockSpec((2 * _BH, _L), lambda i: (0, 0)),
            fwd_spec, fwd_spec, fwd_spec,
            inv_spec, inv_spec, inv_spec,
        ],
        out_specs=[
            pl.BlockSpec((_BH, _TOPK), lambda i: (0, 0)),
            pl.BlockSpec((_BH, _TOPK), lambda i: (0, 0)),
        ],
        out_shape=[
            jax.ShapeDtypeStruct((_BH, _TOPK), jnp.float32),
            jax.ShapeDtypeStruct((_BH, _TOPK), jnp.int32),
        ],
        scratch_shapes=[pltpu.VMEM((_BH, _L), jnp.float32)],
        compiler_params=pltpu.CompilerParams(
            dimension_semantics=("arbitrary",)),
    )(qk, *(jnp.asarray(c) for c in _CFS3),
      *(jnp.asarray(c) for c in _CIS3))

    out24 = pl.pallas_call(
        _combine_body,
        grid_spec=pltpu.PrefetchScalarGridSpec(
            num_scalar_prefetch=2,
            grid=(_BH,),
            in_specs=[pl.BlockSpec((_BH, _L), lambda i, *_: (0, 0))],
            out_specs=pl.BlockSpec((1, 1, _L), lambda i, *_: (i, 0, 0)),
        ),
        out_shape=jax.ShapeDtypeStruct((_BH, 1, _L), jnp.float32),
    )(idx.reshape(-1), wn.reshape(-1), vs)

    return out24.reshape(_B, _H, _L)[..., None]


# speedup vs baseline: 641.5823x; 1.4141x over previous
import functools

import jax
import jax.numpy as jnp
import numpy as np
from jax.experimental import pallas as pl
from jax.experimental.pallas import tpu as pltpu

_L = 2048
_N1 = 2 * _L - 1
_N2 = 2 * _L - 2
_B = 2
_H = 12
_D = 64
_BH = _B * _H
_TOPK = 40
_KB = 256
_HIGHEST = jax.lax.Precision.HIGHEST


def _split3_np(x):
    import ml_dtypes
    bf = ml_dtypes.bfloat16
    x0 = x.astype(bf)
    r = x.astype(np.float32) - x0.astype(np.float32)
    x1 = r.astype(bf)
    x2 = (r - x1.astype(np.float32)).astype(bf)
    return x0, x1, x2


def _dft_constants():
    t = np.arange(_L, dtype=np.int64)
    outer = np.outer(t, t)
    ang1 = (outer % _N1).astype(np.float64) * (2.0 * np.pi / _N1)
    ang2 = (outer % _N2).astype(np.float64) * (2.0 * np.pi / _N2)
    cf = np.cos(ang1).astype(np.float32)
    sf = np.sin(ang1).astype(np.float32)
    ci = np.cos(ang2).astype(np.float32)
    si = np.sin(ang2).astype(np.float32)
    nb = _L // _KB
    cfs = np.stack([np.hstack([cf[:, i * _KB:(i + 1) * _KB],
                               sf[:, i * _KB:(i + 1) * _KB]])
                    for i in range(nb)])
    cis = np.stack([np.vstack([ci[i * _KB:(i + 1) * _KB],
                               si[i * _KB:(i + 1) * _KB]])
                    for i in range(nb)])
    return _split3_np(cfs), _split3_np(cis)


_CFS3, _CIS3 = _dft_constants()


def _split3(x):
    x0 = x.astype(jnp.bfloat16)
    r = x - x0.astype(jnp.float32)
    x1 = r.astype(jnp.bfloat16)
    x2 = (r - x1.astype(jnp.float32)).astype(jnp.bfloat16)
    return x0, x1, x2


def _dot_x6(a3, b3):
    d = functools.partial(jnp.dot, preferred_element_type=jnp.float32)
    a0, a1, a2 = a3
    b0, b1, b2 = b3
    s = d(a0, b2) + d(a1, b1) + d(a2, b0)
    s = s + (d(a0, b1) + d(a1, b0))
    return s + d(a0, b0)


def _proj_body(q_ref, k_ref, v_ref, w_ref, b_ref, o_ref):
    d = functools.partial(jnp.dot, preferred_element_type=jnp.float32)
    w = w_ref[...]
    b = b_ref[...]
    xq = q_ref[0].astype(jnp.bfloat16)
    xk = k_ref[0].astype(jnp.bfloat16)
    xv = v_ref[0].astype(jnp.bfloat16)
    o_ref[0, :, 0:_H] = d(xq, w[:, 0:_H]) + b[:, 0:_H]
    o_ref[0, :, _H:2 * _H] = d(xk, w[:, _H:2 * _H]) + b[:, _H:2 * _H]
    o_ref[0, :, 2 * _H:3 * _H] = d(xv, w[:, 2 * _H:3 * _H]) + b[:, 2 * _H:3 * _H]


def _score_body(qk_ref, cfs0_ref, cfs1_ref, cfs2_ref, cis0_ref, cis1_ref,
                cis2_ref, wn_ref, idx_ref, acc_ref):
    i = pl.program_id(0)

    @pl.when(i == 0)
    def _():
        acc_ref[...] = jnp.zeros_like(acc_ref)

    qk3 = _split3(qk_ref[...])
    reim = _dot_x6(qk3, (cfs0_ref[0], cfs1_ref[0], cfs2_ref[0]))
    re = reim[:, :_KB]
    im = -reim[:, _KB:]
    qr, kr = re[:_BH], re[_BH:]
    qi, ki = im[:_BH], im[_BH:]
    sr = qr * kr + qi * ki
    si = qi * kr - qr * ki
    gcol = i * _KB + jax.lax.broadcasted_iota(jnp.int32, (_BH, _KB), 1)
    cr = jnp.where((gcol == 0) | (gcol == _L - 1), 1.0, 2.0)
    cim = jnp.where(gcol == _L - 1, 0.0, cr)
    a2 = jnp.concatenate([sr * cr, -(si * cim)], axis=1)
    acc_ref[...] += _dot_x6(_split3(a2),
                            (cis0_ref[0], cis1_ref[0], cis2_ref[0]))

    @pl.when(i == pl.num_programs(0) - 1)
    def _():
        score = acc_ref[...] * (1.0 / (float(_N2) * float(_L)))
        iota = jax.lax.broadcasted_iota(jnp.int32, (_BH, _L), 1)
        vals = []
        idxs = []
        for _d in range(_TOPK):
            m = jnp.max(score, axis=1, keepdims=True)
            hit = score == m
            idx = jnp.min(jnp.where(hit, iota, _L), axis=1, keepdims=True)
            vals.append(m)
            idxs.append(idx)
            score = jnp.where(iota == idx, -jnp.inf, score)
        w = jnp.concatenate(vals, axis=1)
        e = jnp.exp(w - w[:, 0:1])
        wn_ref[...] = e / jnp.sum(e, axis=1, keepdims=True)
        idx_ref[...] = jnp.concatenate(idxs, axis=1)


def _combine_body(i_smem, w_smem, vs_ref, out_ref):
    row = pl.program_id(0)
    vrow = vs_ref[pl.ds(row, 1), :]
    acc = jnp.zeros((1, _L), jnp.float32)
    for d in range(_TOPK):
        st = i_smem[row * _TOPK + d]
        w = w_smem[row * _TOPK + d]
        acc = acc + pltpu.roll(vrow, -st, axis=1) * w
    out_ref[0] = acc


@jax.jit
def kernel(queries, keys, values, Wq, bq, Wk, bk, Wv, bv):
    from jax.scipy.linalg import block_diag
    qf = queries.reshape(_B, _L, _H * _D)
    kf = keys.reshape(_B, _L, _H * _D)
    vf = values.reshape(_B, _L, _H * _D)
    wb = jnp.concatenate(
        [block_diag(*([w[0][:, None]] * _H)) for w in (Wq, Wk, Wv)],
        axis=1).astype(jnp.bfloat16)
    bb = jnp.concatenate([jnp.repeat(b, _H) for b in (bq, bk, bv)])[None, :]

    sig = pl.pallas_call(
        _proj_body,
        grid=(_B,),
        in_specs=[
            pl.BlockSpec((1, _L, _H * _D), lambda i: (i, 0, 0)),
            pl.BlockSpec((1, _L, _H * _D), lambda i: (i, 0, 0)),
            pl.BlockSpec((1, _L, _H * _D), lambda i: (i, 0, 0)),
            pl.BlockSpec((_H * _D, 3 * _H), lambda i: (0, 0)),
            pl.BlockSpec((1, 3 * _H), lambda i: (0, 0)),
        ],
        out_specs=pl.BlockSpec((1, _L, 3 * _H), lambda i: (i, 0, 0)),
        out_shape=jax.ShapeDtypeStruct((_B, _L, 3 * _H), jnp.float32),
    )(qf, kf, vf, wb, bb)
    s = sig.transpose(0, 2, 1)
    qk = jnp.concatenate([s[:, 0:_H].reshape(_BH, _L),
                          s[:, _H:2 * _H].reshape(_BH, _L)], axis=0)
    vs = s[:, 2 * _H:3 * _H].reshape(_BH, _L)

    nsteps = _L // _KB
    fwd_spec = pl.BlockSpec((1, _L, 2 * _KB), lambda i: (i, 0, 0))
    inv_spec = pl.BlockSpec((1, 2 * _KB, _L), lambda i: (i, 0, 0))
    wn, idx = pl.pallas_call(
        _score_body,
        grid=(nsteps,),
        in_specs=[
            pl.BlockSpec((2 * _BH, _L), lambda i: (0, 0)),
            fwd_spec, fwd_spec, fwd_spec,
            inv_spec, inv_spec, inv_spec,
        ],
        out_specs=[
            pl.BlockSpec((_BH, _TOPK), lambda i: (0, 0)),
            pl.BlockSpec((_BH, _TOPK), lambda i: (0, 0)),
        ],
        out_shape=[
            jax.ShapeDtypeStruct((_BH, _TOPK), jnp.float32),
            jax.ShapeDtypeStruct((_BH, _TOPK), jnp.int32),
        ],
        scratch_shapes=[pltpu.VMEM((_BH, _L), jnp.float32)],
        compiler_params=pltpu.CompilerParams(
            dimension_semantics=("arbitrary",)),
    )(qk, *(jnp.asarray(c) for c in _CFS3),
      *(jnp.asarray(c) for c in _CIS3))

    out24 = pl.pallas_call(
        _combine_body,
        grid_spec=pltpu.PrefetchScalarGridSpec(
            num_scalar_prefetch=2,
            grid=(_BH,),
            in_specs=[pl.BlockSpec((_BH, _L), lambda i, *_: (0, 0))],
            out_specs=pl.BlockSpec((1, 1, _L), lambda i, *_: (i, 0, 0)),
        ),
        out_shape=jax.ShapeDtypeStruct((_BH, 1, _L), jnp.float32),
    )(idx.reshape(-1), wn.reshape(-1), vs)

    return out24.reshape(_B, _H, _L)[..., None]
